# Initial kernel scaffold; baseline (speedup 1.0000x reference)
import jax
import jax.numpy as jnp
from jax.experimental import pallas as pl


def kernel(x, p, sensors, z):
    raise NotImplementedError("write your pallas kernel here")


# speedup vs baseline: 1.1149x; 1.1149x over previous
import jax
import jax.numpy as jnp
from jax.experimental import pallas as pl

P = 1048576
S = 32
M = 8
MAP_SIZE = 32.0
SENSOR_NOISE = 0.1
EPS_ROUGH = 0.01
EFF = 0.25
DENOM = 2.0 * (SENSOR_NOISE ** 2 + EPS_ROUGH ** 2)

BLK = 4096


def _dense_body(sens_ref, z_ref, xt_ref, pt_ref, pn_ref):
    pos2 = xt_ref[0:2, :]
    px = xt_ref[0:1, :]
    py = xt_ref[1:2, :]
    x2 = px * px + py * py
    sens = sens_ref[...]
    s2 = jnp.sum(sens * sens, axis=1, keepdims=True)
    mm = jax.lax.dot_general(
        sens, pos2, (((1,), (0,)), ((), ())),
        preferred_element_type=jnp.float32)
    d2 = jnp.maximum(x2 + s2 - 2.0 * mm, 0.0)

    m1 = jnp.minimum(d2[0:16], d2[16:32])
    m2 = jnp.maximum(d2[0:16], d2[16:32])
    h = 8
    while h >= 1:
        a1, a2 = m1[0:h], m2[0:h]
        b1, b2 = m1[h:2 * h], m2[h:2 * h]
        nm1 = jnp.minimum(a1, b1)
        nm2 = jnp.minimum(jnp.maximum(a1, b1), jnp.minimum(a2, b2))
        m1, m2 = nm1, nm2
        h //= 2
    d0 = jnp.sqrt(m1)
    d1 = jnp.sqrt(m2)
    z0 = z_ref[0, 0]
    z1 = z_ref[0, 1]
    f0 = d0 - z0
    f1 = d1 - z1
    lw = -(f0 * f0 + f1 * f1) / DENOM
    pn_ref[...] = pt_ref[...] + lw


def kernel(x, p, sensors, z):
    xt = x.T
    pt = p.T
    z2 = z[:2].reshape(1, 2)

    grid = (P // BLK,)
    pn_t = pl.pallas_call(
        _dense_body,
        grid=grid,
        in_specs=[
            pl.BlockSpec((S, 2), lambda i: (0, 0)),
            pl.BlockSpec((1, 2), lambda i: (0, 0)),
            pl.BlockSpec((3, BLK), lambda i: (0, i)),
            pl.BlockSpec((1, BLK), lambda i: (0, i)),
        ],
        out_specs=pl.BlockSpec((1, BLK), lambda i: (0, i)),
        out_shape=jax.ShapeDtypeStruct((1, P), jnp.float32),
    )(sensors, z2, xt, pt)

    p_new = pn_t.T

    logp = jax.nn.log_softmax(p_new, axis=0)
    probs = jnp.exp(logp)
    n_eff = 1.0 / jnp.sum(probs ** 2)
    cond = n_eff <= (EFF * P)
    cdf = jnp.cumsum(probs[:, 0])
    u = jax.random.uniform(jax.random.key(42), (P,), dtype=jnp.float32)
    idx = jnp.clip(jnp.searchsorted(cdf, u, side='right'), 0, P - 1)
    x_res = jnp.take(x, idx, axis=0)
    p_uni = jnp.full((P, 1), jnp.log(1.0 / P), dtype=p.dtype)
    x_out = jnp.where(cond, x_res, x)
    p_out = jnp.where(cond, p_uni, logp)
    return jnp.concatenate([x_out, p_out], axis=1)


# speedup vs baseline: 5.8203x; 5.2204x over previous
import functools
import math

import jax
import jax.numpy as jnp
from jax import lax
from jax.experimental import pallas as pl
from jax.experimental.pallas import tpu as pltpu
from jax.experimental.pallas import tpu_sc as plsc

P = 1048576
S = 32
MAP_SIZE = 32.0
SENSOR_NOISE = 0.1
EPS_ROUGH = 0.01
EFF = 0.25
DENOM = 2.0 * (SENSOR_NOISE ** 2 + EPS_ROUGH ** 2)
LOG_UNI = math.log(1.0 / P)

BLK = 4096
ROWS = P // BLK
NW = 32
CHUNK = P // NW
NT16 = P // 16
T16W = CHUNK // 16
TILE = 1024
L = 16


def _dense_body(sens_ref, z_ref, xt_ref, pt_ref, pn_ref, gmax_ref):
    pos2 = xt_ref[0:2, :]
    px = xt_ref[0:1, :]
    py = xt_ref[1:2, :]
    x2 = px * px + py * py
    sens = sens_ref[...]
    s2 = jnp.sum(sens * sens, axis=1, keepdims=True)
    mm = jax.lax.dot_general(
        sens, pos2, (((1,), (0,)), ((), ())),
        preferred_element_type=jnp.float32)
    d2 = jnp.maximum(x2 + s2 - 2.0 * mm, 0.0)

    m1 = jnp.minimum(d2[0:16], d2[16:32])
    m2 = jnp.maximum(d2[0:16], d2[16:32])
    h = 8
    while h >= 1:
        a1, a2 = m1[0:h], m2[0:h]
        b1, b2 = m1[h:2 * h], m2[h:2 * h]
        nm1 = jnp.minimum(a1, b1)
        nm2 = jnp.minimum(jnp.maximum(a1, b1), jnp.minimum(a2, b2))
        m1, m2 = nm1, nm2
        h //= 2
    d0 = jnp.sqrt(m1)
    d1 = jnp.sqrt(m2)
    f0 = d0 - z_ref[0, 0]
    f1 = d1 - z_ref[0, 1]
    lw = -(f0 * f0 + f1 * f1) / DENOM
    pn = pt_ref[...] + lw
    pn_ref[...] = pn.reshape(1, 1, BLK)

    i = pl.program_id(0)
    bmax = jnp.max(pn)

    @pl.when(i == 0)
    def _():
        gmax_ref[0, 0] = bmax

    @pl.when(i > 0)
    def _():
        gmax_ref[0, 0] = jnp.maximum(gmax_ref[0, 0], bmax)


def _dense_call(xt, pt, sensors, z2):
    return pl.pallas_call(
        _dense_body,
        grid=(ROWS,),
        in_specs=[
            pl.BlockSpec((S, 2), lambda i: (0, 0)),
            pl.BlockSpec((1, 2), lambda i: (0, 0)),
            pl.BlockSpec((3, BLK), lambda i: (0, i)),
            pl.BlockSpec((1, BLK), lambda i: (0, i)),
        ],
        out_specs=[
            pl.BlockSpec((1, 1, BLK), lambda i: (i, 0, 0)),
            pl.BlockSpec((1, 1), lambda i: (0, 0),
                         memory_space=pltpu.SMEM),
        ],
        out_shape=[
            jax.ShapeDtypeStruct((ROWS, 1, BLK), jnp.float32),
            jax.ShapeDtypeStruct((1, 1), jnp.float32),
        ],
    )(sensors, z2, xt, pt)


def _cdf_stage_kernel(cdf1):
    mesh = plsc.VectorSubcoreMesh(core_axis_name="c", subcore_axis_name="s")

    @functools.partial(
        pl.kernel, mesh=mesh,
        out_type=[
            jax.ShapeDtypeStruct((NT16, 16), jnp.float32),
            jax.ShapeDtypeStruct((NT16,), jnp.float32),
        ],
        scratch_types=[
            pltpu.VMEM((CHUNK,), jnp.float32),
            pltpu.VMEM((T16W, 16), jnp.float32),
            pltpu.VMEM((T16W,), jnp.float32),
            pltpu.SemaphoreType.DMA,
        ],
        compiler_params=pltpu.CompilerParams(needs_layout_passes=False, use_tc_tiling_on_sc=False),
    )
    def k(cdf_hbm_in, cdf_hbm, t16_hbm, cbuf1, cbuf, tbuf, sem):
        wid = lax.axis_index("s") * 2 + lax.axis_index("c")
        base = wid * CHUNK
        pltpu.sync_copy(cdf_hbm_in.at[pl.ds(base, CHUNK)], cbuf1)
        lane = lax.iota(jnp.int32, 16)

        def rbody(i, _):
            cbuf[i, :] = cbuf1[pl.ds(i * 16, 16)]
            return 0

        lax.fori_loop(0, T16W, rbody, 0)

        def tbody(i, _):
            idx = (i * 16 + lane) * 16 + 15
            tbuf[pl.ds(i * 16, 16)] = plsc.load_gather(cbuf1, [idx])
            return 0

        lax.fori_loop(0, T16W // 16, tbody, 0)

        pltpu.sync_copy(cbuf, cdf_hbm.at[pl.ds(wid * T16W, T16W)])
        pltpu.sync_copy(tbuf, t16_hbm.at[pl.ds(wid * T16W, T16W)])

    return k(cdf1)


XSUB = 4096


def _reformat_kernel(x1d):
    mesh = plsc.VectorSubcoreMesh(core_axis_name="c", subcore_axis_name="s")

    @functools.partial(
        pl.kernel, mesh=mesh,
        out_type=jax.ShapeDtypeStruct((P, 16), jnp.float32),
        scratch_types=[
            pltpu.VMEM((XSUB * 3,), jnp.float32),
            pltpu.VMEM((XSUB, 16), jnp.float32),
            pltpu.SemaphoreType.DMA,
        ],
        compiler_params=pltpu.CompilerParams(needs_layout_passes=False, use_tc_tiling_on_sc=False),
    )
    def k(x_hbm, xsc_hbm, ibuf, obuf, sem):
        wid = lax.axis_index("s") * 2 + lax.axis_index("c")
        lane = lax.iota(jnp.int32, 16)

        def tile_body(t, _):
            base = wid * CHUNK + t * XSUB
            pltpu.sync_copy(x_hbm.at[pl.ds(base * 3, XSUB * 3)], ibuf)

            def p(v, _):
                rows = v * 16 + lane
                for c in range(3):
                    vals = plsc.load_gather(ibuf, [rows * 3 + c])
                    plsc.store_scatter(
                        obuf, [rows, jnp.full((16,), c, jnp.int32)], vals)
                plsc.store_scatter(
                    obuf, [rows, jnp.full((16,), 3, jnp.int32)],
                    jnp.full((16,), LOG_UNI, jnp.float32))
                return 0

            lax.fori_loop(0, XSUB // 16, p, 0, unroll=2)
            pltpu.sync_copy(obuf, xsc_hbm.at[pl.ds(base, XSUB)])
            return 0

        lax.fori_loop(0, CHUNK // XSUB, tile_body, 0)

    return k(x1d)


def _resample_kernel(u, cdf, t16, xsc):
    mesh = plsc.VectorSubcoreMesh(core_axis_name="c", subcore_axis_name="s")

    @functools.partial(
        pl.kernel, mesh=mesh,
        out_type=jax.ShapeDtypeStruct((P, 4), jnp.float32),
        scratch_types=[
            pltpu.VMEM((NT16,), jnp.float32),
            pltpu.VMEM((TILE,), jnp.float32),
            pltpu.VMEM((TILE,), jnp.int32),
            pltpu.VMEM((TILE, 16), jnp.float32),
            pltpu.VMEM((TILE,), jnp.int32),
            pltpu.VMEM((TILE, 16), jnp.float32),
            pltpu.VMEM((TILE, 4), jnp.float32),
            pltpu.SemaphoreType.DMA,
        ],
        compiler_params=pltpu.CompilerParams(needs_layout_passes=False, use_tc_tiling_on_sc=False),
    )
    def k(u_hbm, cdf_hbm, t16_hbm, xsc_hbm, out_hbm,
          tab, ubuf, jbuf, chunk, ibuf, gbuf, obuf, sem):
        wid = lax.axis_index("s") * 2 + lax.axis_index("c")
        pltpu.sync_copy(t16_hbm, tab)
        lane = lax.iota(jnp.int32, 16)

        def tile_body(t, _):
            base = wid * CHUNK + t * TILE
            pltpu.sync_copy(u_hbm.at[pl.ds(base, TILE)], ubuf)

            def p1(v, _):
                uv = ubuf[pl.ds(v * 16, 16)]
                pos = jnp.zeros((16,), jnp.int32)
                for s in range(15, -1, -1):
                    step = 1 << s
                    probe = pos + (step - 1)
                    val = plsc.load_gather(tab, [probe])
                    pos = jnp.where(val <= uv, pos + step, pos)
                jbuf[pl.ds(v * 16, 16)] = jnp.minimum(pos, NT16 - 1)
                return 0

            lax.fori_loop(0, TILE // 16, p1, 0, unroll=2)

            pltpu.async_copy(cdf_hbm.at[jbuf], chunk, sem).wait()

            def p2(v, _):
                uv = ubuf[pl.ds(v * 16, 16)]
                j16 = jbuf[pl.ds(v * 16, 16)]
                rows = v * 16 + lane
                pos = jnp.zeros((16,), jnp.int32)
                for s in (8, 4, 2, 1):
                    val = plsc.load_gather(chunk, [rows, pos + (s - 1)])
                    pos = jnp.where(val <= uv, pos + s, pos)
                val = plsc.load_gather(chunk, [rows, pos])
                cnt = pos + jnp.where(val <= uv, 1, 0)
                idx = j16 * 16 + cnt
                ibuf[pl.ds(v * 16, 16)] = jnp.minimum(idx, P - 1)
                return 0

            lax.fori_loop(0, TILE // 16, p2, 0, unroll=2)

            pltpu.async_copy(xsc_hbm.at[ibuf], gbuf, sem).wait()

            def p3(v, _):
                rows = v * 16 + lane
                for c in range(4):
                    cc = jnp.full((16,), c, jnp.int32)
                    plsc.store_scatter(
                        obuf, [rows, cc], plsc.load_gather(gbuf, [rows, cc]))
                return 0

            lax.fori_loop(0, TILE // 16, p3, 0, unroll=2)
            pltpu.sync_copy(obuf, out_hbm.at[pl.ds(base, TILE)])
            return 0

        lax.fori_loop(0, CHUNK // TILE, tile_body, 0)

    return k(u, cdf, t16, xsc)


def kernel(x, p, sensors, z):
    xt = x.T
    pt = p.T
    z2 = z[:2].reshape(1, 2)

    pn, gmax = _dense_call(xt, pt, sensors, z2)
    p_new = pn.reshape(P, 1)

    logp = jax.nn.log_softmax(p_new, axis=0)
    probs = jnp.exp(logp)
    n_eff = 1.0 / jnp.sum(probs ** 2)
    cond = n_eff <= (EFF * P)
    cdf1 = jnp.cumsum(probs[:, 0])

    cdf, t16 = _cdf_stage_kernel(cdf1)

    u = jax.random.uniform(jax.random.key(42), (P,), dtype=jnp.float32)

    def resample(_):
        xsc = _reformat_kernel(x.reshape(P * 3))
        return _resample_kernel(u, cdf, t16, xsc)

    def no_resample(_):
        return jnp.concatenate([x, logp], axis=1)

    return lax.cond(cond, resample, no_resample, operand=None)
